# Initial kernel scaffold; baseline (speedup 1.0000x reference)
import jax
import jax.numpy as jnp
from jax.experimental import pallas as pl


def kernel(product_groups, color_groups, index_name, product_group_table, color_group_table, index_name_table):
    raise NotImplementedError("write your pallas kernel here")


# speedup vs baseline: 3.7263x; 3.7263x over previous
import jax
import jax.numpy as jnp
from jax import lax
from jax.experimental import pallas as pl
from jax.experimental.pallas import tpu as pltpu
from jax.experimental.pallas import tpu_sc as plsc

_B = 16384
_PG_D, _CG_D, _IN_D = 64, 128, 32
_OUT_D = _PG_D + _CG_D + _IN_D


def _build(nc, ns):
    nw = nc * ns
    bpw = _B // nw

    def body(pg_idx, cg_idx, in_idx, pg_tab, cg_tab, in_tab, out,
             pgi_v, cgi_v, ini_v, pg_v, cg_v, in_v, sem):
        wid = lax.axis_index("s") * nc + lax.axis_index("c")
        base = wid * bpw
        c0 = pltpu.async_copy(pg_idx.at[pl.ds(base, bpw)], pgi_v, sem)
        c1 = pltpu.async_copy(cg_idx.at[pl.ds(base, bpw)], cgi_v, sem)
        c2 = pltpu.async_copy(in_idx.at[pl.ds(base, bpw)], ini_v, sem)
        c0.wait()
        c1.wait()
        c2.wait()
        g0 = pltpu.async_copy(pg_tab.at[pgi_v], pg_v, sem)
        g1 = pltpu.async_copy(cg_tab.at[cgi_v], cg_v, sem)
        g2 = pltpu.async_copy(in_tab.at[ini_v], in_v, sem)
        g0.wait()
        g1.wait()
        g2.wait()
        s0 = pltpu.async_copy(pg_v, out.at[pl.ds(base, bpw), pl.ds(0, _PG_D)], sem)
        s1 = pltpu.async_copy(cg_v, out.at[pl.ds(base, bpw), pl.ds(_PG_D, _CG_D)], sem)
        s2 = pltpu.async_copy(in_v, out.at[pl.ds(base, bpw), pl.ds(_PG_D + _CG_D, _IN_D)], sem)
        s0.wait()
        s1.wait()
        s2.wait()

    mesh = plsc.VectorSubcoreMesh(core_axis_name="c", subcore_axis_name="s")
    return pl.kernel(
        body,
        out_type=jax.ShapeDtypeStruct((_B, _OUT_D), jnp.float32),
        mesh=mesh,
        compiler_params=pltpu.CompilerParams(use_tc_tiling_on_sc=False),
        scratch_types=[
            pltpu.VMEM((bpw,), jnp.int32),
            pltpu.VMEM((bpw,), jnp.int32),
            pltpu.VMEM((bpw,), jnp.int32),
            pltpu.VMEM((bpw, _PG_D), jnp.float32),
            pltpu.VMEM((bpw, _CG_D), jnp.float32),
            pltpu.VMEM((bpw, _IN_D), jnp.float32),
            pltpu.SemaphoreType.DMA,
        ],
    )


def kernel(product_groups, color_groups, index_name,
           product_group_table, color_group_table, index_name_table):
    info = plsc.get_sparse_core_info()
    k = _build(info.num_cores, info.num_subcores)
    return k(product_groups.astype(jnp.int32),
             color_groups.astype(jnp.int32),
             index_name.astype(jnp.int32),
             product_group_table, color_group_table, index_name_table)
